# Initial kernel scaffold; baseline (speedup 1.0000x reference)
import jax
import jax.numpy as jnp
from jax.experimental import pallas as pl


def kernel(x, edge_index, Wre0, Wre1, Wre2, Wre3, Wre4, Wim0, Wim1, Wim2, Wim3, Wim4, Wl, Wr, b):
    raise NotImplementedError("write your pallas kernel here")


# speedup vs baseline: 3.2788x; 3.2788x over previous
import functools

import jax
import jax.numpy as jnp
from jax import lax
from jax.experimental import pallas as pl
from jax.experimental.pallas import tpu as pltpu
from jax.experimental.pallas import tpu_sc as plsc

NSUB = 16
NCORE = 2
CH = 128
LANES = 16
TRASH_PAD = 64


def _ceil_to(a, m):
    return (a + m - 1) // m * m


def _build_prep(NP, EP):
    EW = EP // NSUB
    NR = NP // NSUB
    NCHW = EW // CH
    HNP = NP // NCORE
    CV = CH // LANES
    mesh = plsc.VectorSubcoreMesh(core_axis_name="c", subcore_axis_name="s")

    @functools.partial(
        pl.kernel,
        out_type=(jax.ShapeDtypeStruct((NP,), jnp.float32),
                  jax.ShapeDtypeStruct((NP,), jnp.float32),
                  jax.ShapeDtypeStruct((NCORE, NSUB, NCHW, CH), jnp.int32)),
        mesh=mesh,
        compiler_params=pltpu.CompilerParams(needs_layout_passes=False),
        scratch_types=[
            pltpu.VMEM((EW,), jnp.int32),
            pltpu.VMEM((NCHW, CH), jnp.int32),
            pltpu.VMEM((NP,), jnp.float32),
            pltpu.VMEM_SHARED((NSUB, NP), jnp.float32),
            pltpu.VMEM((NR,), jnp.float32),
            pltpu.VMEM((NR,), jnp.float32),
            pltpu.VMEM((NR,), jnp.float32),
        ],
    )
    def prep(dst_hbm, dinv_hbm, dinv2_hbm, dloc_hbm,
             dst_v, dloc_v, degloc, shp, accs, tmps, d2s):
        wid = lax.axis_index("s")
        core = lax.axis_index("c")
        zero16 = jnp.zeros((LANES,), jnp.float32)

        pltpu.sync_copy(dst_hbm.at[pl.ds(wid * EW, EW)], dst_v)

        lo = core * HNP

        def remap(r, _):
            for c in range(CV):
                d = dst_v[pl.ds(r * CH + c * LANES, LANES)]
                dl = d - lo
                ok = (dl >= 0) & (dl < HNP)
                dloc_v[r, pl.ds(c * LANES, LANES)] = jnp.where(ok, dl, HNP)
            return 0
        lax.fori_loop(0, NCHW, remap, 0)
        pltpu.sync_copy(dloc_v, dloc_hbm.at[core, wid])

        def zloc(i, _):
            degloc[pl.ds(i * LANES, LANES)] = zero16
            return 0
        lax.fori_loop(0, NP // LANES, zloc, 0)

        ones = jnp.ones((LANES,), jnp.float32)

        def scat(i, _):
            idx = dst_v[pl.ds(i * LANES, LANES)]
            plsc.addupdate_scatter(degloc, [idx], ones)
            return 0
        lax.fori_loop(0, EW // LANES, scat, 0)

        pltpu.sync_copy(degloc, shp.at[wid])
        plsc.subcore_barrier()

        def zacc(i, _):
            accs[pl.ds(i * LANES, LANES)] = zero16
            return 0
        lax.fori_loop(0, NR // LANES, zacc, 0)

        for j in range(NSUB):
            pltpu.sync_copy(shp.at[j, pl.ds(wid * NR, NR)], tmps)

            def addv(i, _):
                s = pl.ds(i * LANES, LANES)
                accs[s] = accs[s] + tmps[s]
                return 0
            lax.fori_loop(0, NR // LANES, addv, 0)

        def elemw(i, _):
            s = pl.ds(i * LANES, LANES)
            d = jnp.maximum(accs[s], 1.0)
            d2s[s] = 1.0 / d
            ii = plsc.bitcast(d, jnp.int32)
            ii = jnp.int32(0x5F3759DF) - (ii >> 1)
            y = plsc.bitcast(ii, jnp.float32)
            for _ in range(3):
                y = y * (1.5 - 0.5 * d * y * y)
            accs[s] = y
            return 0
        lax.fori_loop(0, NR // LANES, elemw, 0)

        @pl.when(core == 0)
        def _():
            pltpu.sync_copy(accs, dinv_hbm.at[pl.ds(wid * NR, NR)])
            pltpu.sync_copy(d2s, dinv2_hbm.at[pl.ds(wid * NR, NR)])

    return prep


def _build_adj_apply(NP, EP, D, n_comp):
    EW = EP // NSUB
    NCHW = EW // CH
    HNP = NP // NCORE
    NH = HNP + TRASH_PAD
    NR = HNP // NSUB
    mesh = plsc.VectorSubcoreMesh(core_axis_name="c", subcore_axis_name="s")

    @functools.partial(
        pl.kernel,
        out_type=tuple(jax.ShapeDtypeStruct((NP, D), jnp.float32)
                       for _ in range(n_comp)),
        mesh=mesh,
        compiler_params=pltpu.CompilerParams(needs_layout_passes=False),
        scratch_types=[
            pltpu.VMEM((NCHW, CH), jnp.int32),
            pltpu.VMEM((NCHW, CH), jnp.int32),
            pltpu.VMEM((CH, D), jnp.float32),
            pltpu.VMEM_SHARED((NH, D), jnp.float32),
            pltpu.VMEM((64, D), jnp.float32),
            pltpu.SemaphoreType.DMA,
        ],
    )
    def adj(*refs):
        v_hbms = refs[2:2 + n_comp]
        s_os = refs[2 + n_comp:2 + 2 * n_comp]
        src_hbm, dloc_hbm = refs[0], refs[1]
        src_v, dst_v, rows0, acc, zbuf, sem0 = refs[2 + 2 * n_comp:]
        wid = lax.axis_index("s")
        core = lax.axis_index("c")
        zero16 = jnp.zeros((LANES,), jnp.float32)

        def zz(i, _):
            for f in range(D // LANES):
                zbuf[i, pl.ds(f * LANES, LANES)] = zero16
            return 0
        lax.fori_loop(0, 64, zz, 0)

        pltpu.sync_copy(src_hbm.at[wid], src_v)
        pltpu.sync_copy(dloc_hbm.at[core, wid], dst_v)

        def comp(v_hbm, s_hbm):
            def zacc(k, _):
                pltpu.sync_copy(zbuf, acc.at[pl.ds(wid * NR + k * 64, 64)])
                return 0
            lax.fori_loop(0, NR // 64, zacc, 0)
            plsc.subcore_barrier()

            def chunk(c, _):
                pltpu.async_copy(v_hbm.at[src_v.at[c]], rows0, sem0).wait()
                pltpu.sync_copy(rows0, acc.at[dst_v.at[c]], add=True)
                return 0
            lax.fori_loop(0, NCHW, chunk, 0)
            plsc.subcore_barrier()

            def wout(k, _):
                loc = pl.ds(wid * NR + k * 64, 64)
                glob = pl.ds(core * HNP + wid * NR + k * 64, 64)
                pltpu.sync_copy(acc.at[loc], s_hbm.at[glob])
                return 0
            lax.fori_loop(0, NR // 64, wout, 0)
            plsc.subcore_barrier()

        for v_hbm, s_hbm in zip(v_hbms, s_os):
            comp(v_hbm, s_hbm)

    return adj


def _build_cmatmul(NP, D, first, resid, BN=1024):
    grid = (NP // BN,)
    rows = pl.BlockSpec((BN, D), lambda i: (i, 0))
    wspec = pl.BlockSpec((D, D), lambda i: (0, 0))
    cols = pl.BlockSpec((BN, 1), lambda i: (i, 0))
    outs = [jax.ShapeDtypeStruct((NP, D), jnp.float32)] * 4

    if first:
        def body(xr_r, wr_r, wi_r, dv_r, yr_o, yi_o, vr_o, vi_o):
            xr = xr_r[...]
            hr = jnp.dot(xr, wr_r[...], preferred_element_type=jnp.float32)
            hi = jnp.dot(xr, wi_r[...], preferred_element_type=jnp.float32)
            dv = dv_r[...]
            yr_o[...] = hr
            yi_o[...] = hi
            vr_o[...] = hr * dv
            vi_o[...] = hi * dv
        in_specs = [rows, wspec, wspec, cols]
    else:
        def body(xr_r, xi_r, wr_r, wi_r, dv_r, yr_o, yi_o, vr_o, vi_o):
            xr = xr_r[...]
            xi = xi_r[...]
            wr = wr_r[...]
            wi = wi_r[...]
            hr = (jnp.dot(xr, wr, preferred_element_type=jnp.float32)
                  - jnp.dot(xi, wi, preferred_element_type=jnp.float32))
            hi = (jnp.dot(xr, wi, preferred_element_type=jnp.float32)
                  + jnp.dot(xi, wr, preferred_element_type=jnp.float32))
            dv = dv_r[...]
            yr_o[...] = hr + (xr if resid else 0.0)
            yi_o[...] = hi + (xi if resid else 0.0)
            vr_o[...] = hr * dv
            vi_o[...] = hi * dv
        in_specs = [rows, rows, wspec, wspec, cols]

    return pl.pallas_call(
        body, grid=grid, in_specs=in_specs,
        out_specs=[rows] * 4, out_shape=outs)


def _build_stepelem(NP, D, inv_t, last, BN=1024):
    grid = (NP // BN,)
    rows = pl.BlockSpec((BN, D), lambda i: (i, 0))
    cols = pl.BlockSpec((BN, 1), lambda i: (i, 0))
    n_out = 2 if last else 4
    outs = [jax.ShapeDtypeStruct((NP, D), jnp.float32)] * n_out

    def body(sr_r, si_r, yr_r, yi_r, dv_r, dv2_r, *out_refs):
        sr = sr_r[...]
        si = si_r[...]
        dv = dv_r[...] * inv_t
        out_refs[0][...] = yr_r[...] - si * dv
        out_refs[1][...] = yi_r[...] + sr * dv
        if not last:
            dv2 = dv2_r[...] * inv_t
            out_refs[2][...] = -si * dv2
            out_refs[3][...] = sr * dv2

    return pl.pallas_call(
        body, grid=grid,
        in_specs=[rows, rows, rows, rows, cols, cols],
        out_specs=[rows] * n_out, out_shape=outs)


def _build_sage(NP, D, BN=1024):
    grid = (NP // BN,)
    rows = pl.BlockSpec((BN, D), lambda i: (i, 0))
    wspec = pl.BlockSpec((D, D), lambda i: (0, 0))
    cols = pl.BlockSpec((BN, 1), lambda i: (i, 0))
    bspec = pl.BlockSpec((1, D), lambda i: (0, 0))

    def body(p_r, y_r, dv2_r, wl_r, wr_r, b_r, o_r):
        mean = p_r[...] * dv2_r[...]
        o_r[...] = (jnp.dot(mean, wl_r[...], preferred_element_type=jnp.float32)
                    + jnp.dot(y_r[...], wr_r[...], preferred_element_type=jnp.float32)
                    + b_r[...])

    return pl.pallas_call(
        body, grid=grid,
        in_specs=[rows, rows, cols, wspec, wspec, bspec],
        out_specs=rows, out_shape=jax.ShapeDtypeStruct((NP, D), jnp.float32))


def kernel(x, edge_index, Wre0, Wre1, Wre2, Wre3, Wre4,
           Wim0, Wim1, Wim2, Wim3, Wim4, Wl, Wr, b):
    N, D = x.shape
    E = edge_index.shape[1]
    T = 4
    NP = _ceil_to(N, NCORE * NSUB * 64)
    EP = _ceil_to(E, NSUB * CH)

    src = edge_index[0]
    dst = edge_index[1]
    if EP > E:
        pad = jnp.full((EP - E,), N, jnp.int32)
        src = jnp.concatenate([src, pad])
        dst = jnp.concatenate([dst, pad])
    src2 = src.reshape(NSUB, EP // (NSUB * CH), CH)
    xp = jnp.pad(x, ((0, NP - N), (0, 0)))

    prep = _build_prep(NP, EP)
    adj = _build_adj_apply(NP, EP, D, n_comp=2)
    aggk = _build_adj_apply(NP, EP, D, n_comp=1)

    dinv, dinv2, dloc = prep(dst)
    dinv_c = dinv.reshape(NP, 1)
    dinv2_c = dinv2.reshape(NP, 1)

    Wres = [Wre0, Wre1, Wre2, Wre3, Wre4]
    Wims = [Wim0, Wim1, Wim2, Wim3, Wim4]
    NL = len(Wres)

    cm_first = _build_cmatmul(NP, D, first=True, resid=False)
    cm_rest = _build_cmatmul(NP, D, first=False, resid=True)
    steps = [_build_stepelem(NP, D, 1.0 / t, last=(t == T))
             for t in range(1, T + 1)]

    yr = yi = None
    for l in range(NL):
        if l == 0:
            yr, yi, vr, vi = cm_first(xp, Wres[0], Wims[0], dinv_c)
        else:
            yr, yi, vr, vi = cm_rest(yr, yi, Wres[l], Wims[l], dinv_c)
        for t in range(1, T + 1):
            sr, si = adj(src2, dloc, vr, vi)
            if t < T:
                yr, yi, vr, vi = steps[t - 1](sr, si, yr, yi, dinv_c, dinv2_c)
            else:
                yr, yi = steps[t - 1](sr, si, yr, yi, dinv_c, dinv2_c)

    (p,) = aggk(src2, dloc, yr)
    out = _build_sage(NP, D)(p, yr, dinv2_c, Wl, Wr, b.reshape(1, D))
    return out[:N]


# speedup vs baseline: 3.9140x; 1.1937x over previous
import functools

import jax
import jax.numpy as jnp
from jax import lax
from jax.experimental import pallas as pl
from jax.experimental.pallas import tpu as pltpu
from jax.experimental.pallas import tpu_sc as plsc

NSUB = 16
NCORE = 2
CH = 128
LANES = 16
TRASH_PAD = 64


def _ceil_to(a, m):
    return (a + m - 1) // m * m


def _build_prep(NP, EP):
    EW = EP // NSUB
    NR = NP // NSUB
    NCHW = EW // CH
    HNP = NP // NCORE
    CV = CH // LANES
    mesh = plsc.VectorSubcoreMesh(core_axis_name="c", subcore_axis_name="s")

    @functools.partial(
        pl.kernel,
        out_type=(jax.ShapeDtypeStruct((NP,), jnp.float32),
                  jax.ShapeDtypeStruct((NP,), jnp.float32),
                  jax.ShapeDtypeStruct((NCORE, NSUB, NCHW, CH), jnp.int32)),
        mesh=mesh,
        compiler_params=pltpu.CompilerParams(needs_layout_passes=False),
        scratch_types=[
            pltpu.VMEM((EW,), jnp.int32),
            pltpu.VMEM((NCHW, CH), jnp.int32),
            pltpu.VMEM((NP,), jnp.float32),
            pltpu.VMEM_SHARED((NSUB, NP), jnp.float32),
            pltpu.VMEM((NR,), jnp.float32),
            pltpu.VMEM((NR,), jnp.float32),
            pltpu.VMEM((NR,), jnp.float32),
        ],
    )
    def prep(dst_hbm, dinv_hbm, dinv2_hbm, dloc_hbm,
             dst_v, dloc_v, degloc, shp, accs, tmps, d2s):
        wid = lax.axis_index("s")
        core = lax.axis_index("c")
        zero16 = jnp.zeros((LANES,), jnp.float32)

        pltpu.sync_copy(dst_hbm.at[pl.ds(wid * EW, EW)], dst_v)

        lo = core * HNP

        def remap(r, _):
            for c in range(CV):
                d = dst_v[pl.ds(r * CH + c * LANES, LANES)]
                dl = d - lo
                ok = (dl >= 0) & (dl < HNP)
                dloc_v[r, pl.ds(c * LANES, LANES)] = jnp.where(ok, dl, HNP)
            return 0
        lax.fori_loop(0, NCHW, remap, 0)
        pltpu.sync_copy(dloc_v, dloc_hbm.at[core, wid])

        def zloc(i, _):
            degloc[pl.ds(i * LANES, LANES)] = zero16
            return 0
        lax.fori_loop(0, NP // LANES, zloc, 0)

        ones = jnp.ones((LANES,), jnp.float32)

        def scat(i, _):
            idx = dst_v[pl.ds(i * LANES, LANES)]
            plsc.addupdate_scatter(degloc, [idx], ones)
            return 0
        lax.fori_loop(0, EW // LANES, scat, 0)

        pltpu.sync_copy(degloc, shp.at[wid])
        plsc.subcore_barrier()

        def zacc(i, _):
            accs[pl.ds(i * LANES, LANES)] = zero16
            return 0
        lax.fori_loop(0, NR // LANES, zacc, 0)

        for j in range(NSUB):
            pltpu.sync_copy(shp.at[j, pl.ds(wid * NR, NR)], tmps)

            def addv(i, _):
                s = pl.ds(i * LANES, LANES)
                accs[s] = accs[s] + tmps[s]
                return 0
            lax.fori_loop(0, NR // LANES, addv, 0)

        def elemw(i, _):
            s = pl.ds(i * LANES, LANES)
            d = jnp.maximum(accs[s], 1.0)
            d2s[s] = 1.0 / d
            ii = plsc.bitcast(d, jnp.int32)
            ii = jnp.int32(0x5F3759DF) - (ii >> 1)
            y = plsc.bitcast(ii, jnp.float32)
            for _ in range(3):
                y = y * (1.5 - 0.5 * d * y * y)
            accs[s] = y
            return 0
        lax.fori_loop(0, NR // LANES, elemw, 0)

        @pl.when(core == 0)
        def _():
            pltpu.sync_copy(accs, dinv_hbm.at[pl.ds(wid * NR, NR)])
            pltpu.sync_copy(d2s, dinv2_hbm.at[pl.ds(wid * NR, NR)])

    return prep


def _build_adj_apply(NP, EP, D, n_comp):
    EW = EP // NSUB
    NCHW = EW // CH
    HNP = NP // NCORE
    NH = HNP + TRASH_PAD
    NR = HNP // NSUB
    mesh = plsc.VectorSubcoreMesh(core_axis_name="c", subcore_axis_name="s")

    @functools.partial(
        pl.kernel,
        out_type=tuple(jax.ShapeDtypeStruct((NP, D), jnp.float32)
                       for _ in range(n_comp)),
        mesh=mesh,
        compiler_params=pltpu.CompilerParams(needs_layout_passes=False),
        scratch_types=[
            pltpu.VMEM((NCHW, CH), jnp.int32),
            pltpu.VMEM((NCHW, CH), jnp.int32),
            pltpu.VMEM((CH, D), jnp.float32),
            pltpu.VMEM((CH, D), jnp.float32),
            pltpu.VMEM_SHARED((NH, D), jnp.float32),
            pltpu.VMEM((64, D), jnp.float32),
            pltpu.SemaphoreType.DMA,
            pltpu.SemaphoreType.DMA,
        ],
    )
    def adj(*refs):
        v_hbms = refs[2:2 + n_comp]
        s_os = refs[2 + n_comp:2 + 2 * n_comp]
        src_hbm, dloc_hbm = refs[0], refs[1]
        src_v, dst_v, rows0, rows1, acc, zbuf, sem0, sem1 = refs[2 + 2 * n_comp:]
        wid = lax.axis_index("s")
        core = lax.axis_index("c")
        zero16 = jnp.zeros((LANES,), jnp.float32)

        def zz(i, _):
            for f in range(D // LANES):
                zbuf[i, pl.ds(f * LANES, LANES)] = zero16
            return 0
        lax.fori_loop(0, 64, zz, 0)

        pltpu.sync_copy(src_hbm.at[wid], src_v)
        pltpu.sync_copy(dloc_hbm.at[core, wid], dst_v)

        def comp(v_hbm, s_hbm):
            def zacc(k, _):
                pltpu.sync_copy(zbuf, acc.at[pl.ds(wid * NR + k * 64, 64)])
                return 0
            lax.fori_loop(0, NR // 64, zacc, 0)
            plsc.subcore_barrier()

            pltpu.async_copy(v_hbm.at[src_v.at[0]], rows0, sem0)

            def pair(p, _):
                c0 = 2 * p
                pltpu.make_async_copy(v_hbm.at[src_v.at[c0]], rows0, sem0).wait()

                @pl.when(c0 + 1 < NCHW)
                def _():
                    pltpu.async_copy(v_hbm.at[src_v.at[c0 + 1]], rows1, sem1)
                pltpu.sync_copy(rows0, acc.at[dst_v.at[c0]], add=True)

                @pl.when(c0 + 1 < NCHW)
                def _():
                    pltpu.make_async_copy(
                        v_hbm.at[src_v.at[c0 + 1]], rows1, sem1).wait()

                    @pl.when(c0 + 2 < NCHW)
                    def _():
                        pltpu.async_copy(
                            v_hbm.at[src_v.at[c0 + 2]], rows0, sem0)
                    pltpu.sync_copy(rows1, acc.at[dst_v.at[c0 + 1]], add=True)
                return 0
            lax.fori_loop(0, (NCHW + 1) // 2, pair, 0)
            plsc.subcore_barrier()

            def wout(k, _):
                loc = pl.ds(wid * NR + k * 64, 64)
                glob = pl.ds(core * HNP + wid * NR + k * 64, 64)
                pltpu.sync_copy(acc.at[loc], s_hbm.at[glob])
                return 0
            lax.fori_loop(0, NR // 64, wout, 0)
            plsc.subcore_barrier()

        for v_hbm, s_hbm in zip(v_hbms, s_os):
            comp(v_hbm, s_hbm)

    return adj


def _build_cmatmul(NP, D, first, resid, BN=1024):
    grid = (NP // BN,)
    rows = pl.BlockSpec((BN, D), lambda i: (i, 0))
    wspec = pl.BlockSpec((D, D), lambda i: (0, 0))
    cols = pl.BlockSpec((BN, 1), lambda i: (i, 0))
    outs = [jax.ShapeDtypeStruct((NP, D), jnp.float32)] * 4

    if first:
        def body(xr_r, wr_r, wi_r, dv_r, yr_o, yi_o, vr_o, vi_o):
            xr = xr_r[...]
            hr = jnp.dot(xr, wr_r[...], preferred_element_type=jnp.float32)
            hi = jnp.dot(xr, wi_r[...], preferred_element_type=jnp.float32)
            dv = dv_r[...]
            yr_o[...] = hr
            yi_o[...] = hi
            vr_o[...] = hr * dv
            vi_o[...] = hi * dv
        in_specs = [rows, wspec, wspec, cols]
    else:
        def body(xr_r, xi_r, wr_r, wi_r, dv_r, yr_o, yi_o, vr_o, vi_o):
            xr = xr_r[...]
            xi = xi_r[...]
            wr = wr_r[...]
            wi = wi_r[...]
            hr = (jnp.dot(xr, wr, preferred_element_type=jnp.float32)
                  - jnp.dot(xi, wi, preferred_element_type=jnp.float32))
            hi = (jnp.dot(xr, wi, preferred_element_type=jnp.float32)
                  + jnp.dot(xi, wr, preferred_element_type=jnp.float32))
            dv = dv_r[...]
            yr_o[...] = hr + (xr if resid else 0.0)
            yi_o[...] = hi + (xi if resid else 0.0)
            vr_o[...] = hr * dv
            vi_o[...] = hi * dv
        in_specs = [rows, rows, wspec, wspec, cols]

    return pl.pallas_call(
        body, grid=grid, in_specs=in_specs,
        out_specs=[rows] * 4, out_shape=outs)


def _build_stepelem(NP, D, inv_t, last, BN=1024):
    grid = (NP // BN,)
    rows = pl.BlockSpec((BN, D), lambda i: (i, 0))
    cols = pl.BlockSpec((BN, 1), lambda i: (i, 0))
    n_out = 2 if last else 4
    outs = [jax.ShapeDtypeStruct((NP, D), jnp.float32)] * n_out

    def body(sr_r, si_r, yr_r, yi_r, dv_r, dv2_r, *out_refs):
        sr = sr_r[...]
        si = si_r[...]
        dv = dv_r[...] * inv_t
        out_refs[0][...] = yr_r[...] - si * dv
        out_refs[1][...] = yi_r[...] + sr * dv
        if not last:
            dv2 = dv2_r[...] * inv_t
            out_refs[2][...] = -si * dv2
            out_refs[3][...] = sr * dv2

    return pl.pallas_call(
        body, grid=grid,
        in_specs=[rows, rows, rows, rows, cols, cols],
        out_specs=[rows] * n_out, out_shape=outs)


def _build_sage(NP, D, BN=1024):
    grid = (NP // BN,)
    rows = pl.BlockSpec((BN, D), lambda i: (i, 0))
    wspec = pl.BlockSpec((D, D), lambda i: (0, 0))
    cols = pl.BlockSpec((BN, 1), lambda i: (i, 0))
    bspec = pl.BlockSpec((1, D), lambda i: (0, 0))

    def body(p_r, y_r, dv2_r, wl_r, wr_r, b_r, o_r):
        mean = p_r[...] * dv2_r[...]
        o_r[...] = (jnp.dot(mean, wl_r[...], preferred_element_type=jnp.float32)
                    + jnp.dot(y_r[...], wr_r[...], preferred_element_type=jnp.float32)
                    + b_r[...])

    return pl.pallas_call(
        body, grid=grid,
        in_specs=[rows, rows, cols, wspec, wspec, bspec],
        out_specs=rows, out_shape=jax.ShapeDtypeStruct((NP, D), jnp.float32))


def kernel(x, edge_index, Wre0, Wre1, Wre2, Wre3, Wre4,
           Wim0, Wim1, Wim2, Wim3, Wim4, Wl, Wr, b):
    N, D = x.shape
    E = edge_index.shape[1]
    T = 4
    NP = _ceil_to(N, NCORE * NSUB * 64)
    EP = _ceil_to(E, NSUB * CH)

    src = edge_index[0]
    dst = edge_index[1]
    if EP > E:
        pad = jnp.full((EP - E,), N, jnp.int32)
        src = jnp.concatenate([src, pad])
        dst = jnp.concatenate([dst, pad])
    src2 = src.reshape(NSUB, EP // (NSUB * CH), CH)
    xp = jnp.pad(x, ((0, NP - N), (0, 0)))

    prep = _build_prep(NP, EP)
    adj = _build_adj_apply(NP, EP, D, n_comp=2)
    aggk = _build_adj_apply(NP, EP, D, n_comp=1)

    dinv, dinv2, dloc = prep(dst)
    dinv_c = dinv.reshape(NP, 1)
    dinv2_c = dinv2.reshape(NP, 1)

    Wres = [Wre0, Wre1, Wre2, Wre3, Wre4]
    Wims = [Wim0, Wim1, Wim2, Wim3, Wim4]
    NL = len(Wres)

    cm_first = _build_cmatmul(NP, D, first=True, resid=False)
    cm_rest = _build_cmatmul(NP, D, first=False, resid=True)
    steps = [_build_stepelem(NP, D, 1.0 / t, last=(t == T))
             for t in range(1, T + 1)]

    yr = yi = None
    for l in range(NL):
        if l == 0:
            yr, yi, vr, vi = cm_first(xp, Wres[0], Wims[0], dinv_c)
        else:
            yr, yi, vr, vi = cm_rest(yr, yi, Wres[l], Wims[l], dinv_c)
        for t in range(1, T + 1):
            sr, si = adj(src2, dloc, vr, vi)
            if t < T:
                yr, yi, vr, vi = steps[t - 1](sr, si, yr, yi, dinv_c, dinv2_c)
            else:
                yr, yi = steps[t - 1](sr, si, yr, yi, dinv_c, dinv2_c)

    (p,) = aggk(src2, dloc, yr)
    out = _build_sage(NP, D)(p, yr, dinv2_c, Wl, Wr, b.reshape(1, D))
    return out[:N]
